# baseline (speedup 1.0000x reference)
import functools

import jax
import jax.numpy as jnp
from jax import lax
from jax.experimental import pallas as pl
from jax.experimental.pallas import tpu as pltpu

_NSTREAM = 4
_SSTEPS = 2


def _gap_head_kernel(*refs, inv_s, nstream, nsteps):
    x_refs = refs[:nstream]
    w_ref, b_ref, out_ref, fea_ref = refs[nstream:]
    s_i = pl.program_id(1)

    part = jnp.sum(x_refs[0][...], axis=1)
    for r in x_refs[1:]:
        part = part + jnp.sum(r[...], axis=1)

    @pl.when(s_i == 0)
    def _():
        fea_ref[...] = part

    @pl.when(s_i != 0)
    def _():
        fea_ref[...] = fea_ref[...] + part

    @pl.when(s_i == nsteps - 1)
    def _():
        fea = fea_ref[...] * inv_s
        fea_ref[...] = fea
        out = lax.dot_general(
            fea, w_ref[...],
            dimension_numbers=(((1,), (1,)), ((), ())),
            preferred_element_type=jnp.float32) + b_ref[...]
        out_ref[...] = out.astype(out_ref.dtype)


def _gap_head(x, weight, bias):
    N, C, D, H, W = x.shape
    S = D * H * W
    out_c = weight.shape[0]

    tn = N if N % 8 else min(N, 8)

    x_nsc = jnp.transpose(x, (0, 2, 3, 4, 1)).reshape(N, S, C)
    b2 = bias.reshape(1, out_c)

    nstream = _NSTREAM if S % (_NSTREAM * _SSTEPS) == 0 else 1
    nsteps = _SSTEPS if S % (_NSTREAM * _SSTEPS) == 0 else 1
    grid = (pl.cdiv(N, tn), nsteps)

    kernel_fn = functools.partial(
        _gap_head_kernel, inv_s=1.0 / S, nstream=nstream, nsteps=nsteps)

    sk = S // (nstream * nsteps)

    def slice_spec(k):
        return pl.BlockSpec((tn, sk, C),
                            lambda n, s, k=k: (n, s * nstream + k, 0))

    out_p, fea_p = pl.pallas_call(
        kernel_fn,
        out_shape=(
            jax.ShapeDtypeStruct((N, out_c), x.dtype),
            jax.ShapeDtypeStruct((N, C), x.dtype),
        ),
        grid=grid,
        in_specs=[slice_spec(k) for k in range(nstream)] + [
            pl.BlockSpec((out_c, C), lambda n, s: (0, 0)),
            pl.BlockSpec((1, out_c), lambda n, s: (0, 0)),
        ],
        out_specs=(
            pl.BlockSpec((tn, out_c), lambda n, s: (n, 0)),
            pl.BlockSpec((tn, C), lambda n, s: (n, 0)),
        ),
        compiler_params=pltpu.CompilerParams(
            dimension_semantics=("arbitrary", "arbitrary"),
            vmem_limit_bytes=60000 * 1024,
        ),
        cost_estimate=pl.CostEstimate(
            flops=N * C * S + 2 * N * C * out_c,
            transcendentals=0,
            bytes_accessed=N * C * S * 4 + (C + 1) * out_c * 4
            + N * (out_c + C) * 4,
        ),
    )(*([x_nsc] * nstream), weight, b2)

    return out_p, fea_p


def kernel(x, weight, bias):
    return _gap_head(x, weight, bias)


# speedup vs baseline: 1.0817x; 1.0817x over previous
import functools

import jax
import jax.numpy as jnp
from jax import lax
from jax.experimental import pallas as pl
from jax.experimental.pallas import tpu as pltpu

_NSTREAM = 8


def _gap_head_kernel(*refs, inv_s, nstream):
    x_refs = refs[:nstream]
    w_ref, b_ref, out_ref, fea_ref = refs[nstream:]

    part = jnp.sum(x_refs[0][...], axis=1)
    for r in x_refs[1:]:
        part = part + jnp.sum(r[...], axis=1)
    fea = part * inv_s
    fea_ref[...] = fea.astype(fea_ref.dtype)

    out = lax.dot_general(
        fea, w_ref[...],
        dimension_numbers=(((1,), (1,)), ((), ())),
        preferred_element_type=jnp.float32) + b_ref[...]
    out_ref[...] = out.astype(out_ref.dtype)


def _gap_head(x, weight, bias):
    N, C, D, H, W = x.shape
    S = D * H * W
    out_c = weight.shape[0]

    tn = N if N % 8 else min(N, 8)
    grid = (pl.cdiv(N, tn),)

    x_nsc = jnp.transpose(x, (0, 2, 3, 4, 1)).reshape(N, S, C)
    b2 = bias.reshape(1, out_c)

    nstream = _NSTREAM if S % _NSTREAM == 0 else 1
    kernel_fn = functools.partial(
        _gap_head_kernel, inv_s=1.0 / S, nstream=nstream)

    sk = S // nstream

    def slice_spec(k):
        return pl.BlockSpec((tn, sk, C), lambda n, k=k: (n, k, 0))

    out_p, fea_p = pl.pallas_call(
        kernel_fn,
        out_shape=(
            jax.ShapeDtypeStruct((N, out_c), x.dtype),
            jax.ShapeDtypeStruct((N, C), x.dtype),
        ),
        grid=grid,
        in_specs=[slice_spec(k) for k in range(nstream)] + [
            pl.BlockSpec((out_c, C), lambda n: (0, 0)),
            pl.BlockSpec((1, out_c), lambda n: (0, 0)),
        ],
        out_specs=(
            pl.BlockSpec((tn, out_c), lambda n: (n, 0)),
            pl.BlockSpec((tn, C), lambda n: (n, 0)),
        ),
        compiler_params=pltpu.CompilerParams(
            dimension_semantics=("arbitrary",),
            vmem_limit_bytes=60000 * 1024,
        ),
        cost_estimate=pl.CostEstimate(
            flops=N * C * S + 2 * N * C * out_c,
            transcendentals=0,
            bytes_accessed=N * C * S * 4 + (C + 1) * out_c * 4
            + N * (out_c + C) * 4,
        ),
    )(*([x_nsc] * nstream), weight, b2)

    return out_p, fea_p


def kernel(x, weight, bias):
    return _gap_head(x, weight, bias)
